# baseline (speedup 1.0000x reference)
import functools

import jax
import jax.numpy as jnp
from jax import lax
from jax.experimental import pallas as pl
from jax.experimental.pallas import tpu as pltpu
from jax.experimental.pallas import tpu_sc as plsc

NC, NS, L = 2, 16, 16
N = 10000
E = 160000
D = 256
H = D // NC
RS = 640
NP = NS * RS
CB = 128
EPT = 10240
NCH = EPT // CB
HCH = NCH // 2
ET = NS * EPT
DCB = 128
EPW = ET // (NC * NS)
DNCH = EPW // DCB

_f32 = jnp.float32
_i32 = jnp.int32


def _deg_body(darr, deg2, degsh, di_all, ones, zb, stg, sem):
    c = lax.axis_index("c")
    s = lax.axis_index("s")
    w = c * NS + s

    for j in range(DCB // L):
        ones[pl.ds(j * L, L)] = jnp.ones((L,), _f32)

    def _z(t, _):
        zb[pl.ds(t * L, L)] = jnp.zeros((L,), _f32)
        return 0

    lax.fori_loop(0, RS // L, _z, 0)
    pltpu.sync_copy(zb, degsh.at[pl.ds(s * RS, RS)])
    pltpu.sync_copy(darr.at[w], di_all)
    plsc.subcore_barrier()

    def _grp(g, _):
        for u in range(8):
            pltpu.async_copy(ones, degsh.at[di_all.at[g * 8 + u]], sem, add=True)
        for u in range(8):
            pltpu.make_async_copy(ones, degsh.at[di_all.at[0]], sem).wait()
        return 0

    lax.fori_loop(0, DNCH // 8, _grp, 0)
    plsc.subcore_barrier()

    pltpu.sync_copy(degsh.at[pl.ds(s * RS, RS)], stg)
    pltpu.sync_copy(stg, deg2.at[pl.ds(c * NP + s * RS, RS)])


_deg_call = functools.partial(
    pl.kernel,
    out_type=jax.ShapeDtypeStruct((NC * NP,), _f32),
    mesh=plsc.VectorSubcoreMesh(core_axis_name="c", subcore_axis_name="s"),
    scratch_types=[
        pltpu.VMEM_SHARED((NP,), _f32),
        pltpu.VMEM((DNCH, DCB), _i32),
        pltpu.VMEM((DCB,), _f32),
        pltpu.VMEM((RS,), _f32),
        pltpu.VMEM((RS,), _f32),
        pltpu.SemaphoreType.DMA,
    ],
)(_deg_body)


def _hops_body(s0, garr, sarr, norm2x,
               o1, o2, o3, o4,
               agg, R, gsl, ssl, gA, gB, sA, sB):
    c = lax.axis_index("c")
    s = lax.axis_index("s")
    w = c * NS + s
    coff = c * NP
    row0 = s * RS

    gsem = [gA, gB]
    ssem = [sA, sB]

    def one_hop(sin, sout):
        def g_start(lc, b):
            pltpu.async_copy(sin.at[gsl.at[lc]], R.at[pl.ds(b * CB, CB)],
                             gsem[b])

        def g_wait(b):
            pltpu.make_async_copy(sin.at[gsl.at[0]], R.at[pl.ds(0, CB)],
                                  gsem[b]).wait()

        def s_start(lc, b):
            pltpu.async_copy(R.at[pl.ds(b * CB, CB)], agg.at[ssl.at[lc]],
                             ssem[b], add=True)

        def s_wait(b):
            pltpu.make_async_copy(R.at[pl.ds(0, CB)], agg.at[ssl.at[0]],
                                  ssem[b]).wait()

        def load_slabs(half):
            pltpu.sync_copy(garr.at[pl.ds(w * NCH + half * HCH, HCH)], gsl)
            pltpu.sync_copy(sarr.at[pl.ds(s * NCH + half * HCH, HCH)], ssl)

        def _zrow(r, _):
            for j in range(H // L):
                R[r, pl.ds(j * L, L)] = jnp.zeros((L,), _f32)
            return 0

        lax.fori_loop(0, CB, _zrow, 0)
        for t in range(RS // CB):
            pltpu.async_copy(R.at[pl.ds(0, CB)],
                             agg.at[pl.ds(row0 + t * CB, CB)], sA)
        for t in range(RS // CB):
            pltpu.make_async_copy(R.at[pl.ds(0, CB)],
                                  agg.at[pl.ds(row0, CB)], sA).wait()
        plsc.subcore_barrier()

        for half in range(2):
            load_slabs(half)
            g_start(0, 0)
            g_wait(0)
            g_start(1, 1)
            s_start(0, 0)

            def _pair(t, _):
                for u in range(2):
                    lc = t * 2 + u
                    b = u
                    g_wait(b)
                    s_wait(1 - b)
                    g_start(lc + 1, 1 - b)
                    s_start(lc, b)
                return 0

            g_wait(1)
            s_wait(0)
            g_start(2, 0)
            s_start(1, 1)

            lax.fori_loop(1, HCH // 2 - 1, _pair, 0)

            g_wait(0)
            s_wait(1)
            g_start(HCH - 1, 1)
            s_start(HCH - 2, 0)
            g_wait(1)
            s_wait(0)
            s_start(HCH - 1, 1)
            s_wait(1)
        plsc.subcore_barrier()

        for t in range(RS // CB):
            pltpu.async_copy(agg.at[pl.ds(row0 + t * CB, CB)],
                             R.at[pl.ds(0, CB)], gA)
            pltpu.async_copy(norm2x.at[pl.ds(row0 + t * CB, CB)],
                             R.at[pl.ds(CB, CB)], gB)
            pltpu.make_async_copy(agg.at[pl.ds(row0, CB)],
                                  R.at[pl.ds(0, CB)], gA).wait()
            pltpu.make_async_copy(norm2x.at[pl.ds(row0, CB)],
                                  R.at[pl.ds(CB, CB)], gB).wait()

            def _srow(r, _):
                for j in range(H // L):
                    R[r, pl.ds(j * L, L)] = (
                        R[r, pl.ds(j * L, L)] * R[r + CB, pl.ds(j * L, L)]
                    )
                return 0

            lax.fori_loop(0, CB, _srow, 0)
            pltpu.async_copy(R.at[pl.ds(0, CB)],
                             sout.at[pl.ds(coff + row0 + t * CB, CB)], sA)
            pltpu.make_async_copy(R.at[pl.ds(0, CB)],
                                  sout.at[pl.ds(coff, CB)], sA).wait()

    souts = [o1, o2, o3, o4]
    sins = [s0] + souts[:3]
    for k in range(4):
        one_hop(sins[k], souts[k])


_hops_call = functools.partial(
    pl.kernel,
    out_type=[jax.ShapeDtypeStruct((NC * NP, H), _f32) for _ in range(4)],
    mesh=plsc.VectorSubcoreMesh(core_axis_name="c", subcore_axis_name="s"),
    scratch_types=[
        pltpu.VMEM_SHARED((NP, H), _f32),
        pltpu.VMEM((2 * CB, H), _f32),
        pltpu.VMEM((HCH, CB), _i32),
        pltpu.VMEM((HCH, CB), _i32),
        pltpu.SemaphoreType.DMA,
        pltpu.SemaphoreType.DMA,
        pltpu.SemaphoreType.DMA,
        pltpu.SemaphoreType.DMA,
    ],
)(_hops_body)


def _prep_body(feat_ref, deg2_ref, s0_ref, n2x_ref, rn_ref):
    d = deg2_ref[0:NP, :] + deg2_ref[NP : 2 * NP, :]
    cl = jnp.maximum(d, 1.0)
    norm = lax.rsqrt(cl)
    n2x_ref[...] = jnp.broadcast_to(1.0 / cl, (NP, H))
    rn_ref[...] = jnp.sqrt(cl)
    s0_ref[0:NP, :] = feat_ref[:, 0:H] * norm
    s0_ref[NP : 2 * NP, :] = feat_ref[:, H : 2 * H] * norm


_prep_call = pl.pallas_call(
    _prep_body,
    out_shape=[
        jax.ShapeDtypeStruct((NC * NP, H), _f32),
        jax.ShapeDtypeStruct((NP, H), _f32),
        jax.ShapeDtypeStruct((NP, 1), _f32),
    ],
)


def _mm_body(feat_ref, w0_ref, wh_ref, rn_ref, b_ref, *rest):
    s_refs = rest[:8]
    out_ref = rest[8]
    acc = jnp.zeros((RS, D), _f32)
    for k in range(8):
        for c in range(NC):
            acc = acc + jnp.dot(s_refs[k][c], wh_ref[k, c],
                                preferred_element_type=_f32)
    base = jnp.dot(feat_ref[...], w0_ref[...], preferred_element_type=_f32)
    out_ref[...] = base + rn_ref[...] * acc + b_ref[...]


_mm_call = pl.pallas_call(
    _mm_body,
    grid=(NS,),
    in_specs=[
        pl.BlockSpec((RS, D), lambda i: (i, 0)),
        pl.BlockSpec((D, D), lambda i: (0, 0)),
        pl.BlockSpec((8, NC, H, D), lambda i: (0, 0, 0, 0)),
        pl.BlockSpec((RS, 1), lambda i: (i, 0)),
        pl.BlockSpec((1, D), lambda i: (0, 0)),
    ] + [pl.BlockSpec((NC, RS, H), lambda i: (0, i, 0)) for _ in range(8)],
    out_specs=pl.BlockSpec((RS, D), lambda i: (i, 0)),
    out_shape=jax.ShapeDtypeStruct((NP, D), _f32),
)


def kernel(feat, edge_index, W, b):
    src = edge_index[0].astype(_i32)
    dst = edge_index[1].astype(_i32)

    pad = ET - E
    zpad = jnp.zeros((pad,), _i32)
    npad = jnp.full((pad,), N, _i32)
    dst_g = jnp.concatenate([dst, zpad]).reshape(NS * NCH, CB)
    dst_s = jnp.concatenate([dst, npad]).reshape(NS * NCH, CB)
    src_g = jnp.concatenate([src, zpad]).reshape(NS * NCH, CB)
    src_s = jnp.concatenate([src, npad]).reshape(NS * NCH, CB)
    dst_g2 = jnp.concatenate([dst_g, dst_g + NP], axis=0)
    src_g2 = jnp.concatenate([src_g, src_g + NP], axis=0)
    darr = dst_s.reshape(NC * NS, DNCH, DCB)

    featp = jnp.pad(feat, ((0, NP - N), (0, 0)))

    deg2 = _deg_call(darr)
    s0, n2x, rn = _prep_call(featp, deg2.reshape(NC * NP, 1))

    s_a = _hops_call(s0, dst_g2, src_s, n2x)
    s_b = _hops_call(s_a[3], src_g2, dst_s, n2x)
    s_list = list(s_a) + list(s_b)

    w0 = W[0:D]
    wh = W[D:].reshape(8, NC, H, D)
    b2 = b.reshape(1, D)
    s3d = [sk.reshape(NC, NP, H) for sk in s_list]
    outp = _mm_call(featp, w0, wh, rn, b2, *s3d)
    return outp[:N]


# speedup vs baseline: 1.0999x; 1.0999x over previous
import functools

import jax
import jax.numpy as jnp
from jax import lax
from jax.experimental import pallas as pl
from jax.experimental.pallas import tpu as pltpu
from jax.experimental.pallas import tpu_sc as plsc

NC, NS, L = 2, 16, 16
N = 10000
E = 160000
D = 256
H = D // NC
RS = 640
NP = NS * RS
CB = 32
EPT = 10240
NCH = EPT // CB
NB = 5
NQ = 10
GA = 4
SLAG = NB - GA
IXA = GA + 2
UNR = 10
ET = NS * EPT
DCB = 128
EPW = ET // (NC * NS)
DNCH = EPW // DCB

_f32 = jnp.float32
_i32 = jnp.int32


def _deg_body(darr, deg2, degsh, di_all, ones, zb, stg, sem):
    c = lax.axis_index("c")
    s = lax.axis_index("s")
    w = c * NS + s

    for j in range(DCB // L):
        ones[pl.ds(j * L, L)] = jnp.ones((L,), _f32)

    def _z(t, _):
        zb[pl.ds(t * L, L)] = jnp.zeros((L,), _f32)
        return 0

    lax.fori_loop(0, RS // L, _z, 0)
    pltpu.sync_copy(zb, degsh.at[pl.ds(s * RS, RS)])
    pltpu.sync_copy(darr.at[w], di_all)
    plsc.subcore_barrier()

    def _grp(g, _):
        for u in range(8):
            pltpu.async_copy(ones, degsh.at[di_all.at[g * 8 + u]], sem, add=True)
        for u in range(8):
            pltpu.make_async_copy(ones, degsh.at[di_all.at[0]], sem).wait()
        return 0

    lax.fori_loop(0, DNCH // 8, _grp, 0)
    plsc.subcore_barrier()

    pltpu.sync_copy(degsh.at[pl.ds(s * RS, RS)], stg)
    pltpu.sync_copy(stg, deg2.at[pl.ds(c * NP + s * RS, RS)])


_deg_call = functools.partial(
    pl.kernel,
    out_type=jax.ShapeDtypeStruct((NC * NP,), _f32),
    mesh=plsc.VectorSubcoreMesh(core_axis_name="c", subcore_axis_name="s"),
    scratch_types=[
        pltpu.VMEM_SHARED((NP,), _f32),
        pltpu.VMEM((DNCH, DCB), _i32),
        pltpu.VMEM((DCB,), _f32),
        pltpu.VMEM((RS,), _f32),
        pltpu.VMEM((RS,), _f32),
        pltpu.SemaphoreType.DMA,
    ],
)(_deg_body)


def _hops_body(s0, garr, sarr, norm2x,
               o1, o2,
               agg, *rest):
    rows = list(rest[0:NB])
    ixg = list(rest[NB:NB + NQ])
    ixs = list(rest[NB + NQ:NB + 2 * NQ])
    gsem = list(rest[NB + 2 * NQ:2 * NB + 2 * NQ])
    ssem = list(rest[2 * NB + 2 * NQ:3 * NB + 2 * NQ])
    psem = list(rest[3 * NB + 2 * NQ:3 * NB + 3 * NQ])

    c = lax.axis_index("c")
    s = lax.axis_index("s")
    w = c * NS + s
    coff = c * NP
    row0 = s * RS

    def one_hop(sin, sout):
        def ix_start(i, q):
            pltpu.async_copy(garr.at[w * NCH + i], ixg[q], psem[q])
            pltpu.async_copy(sarr.at[s * NCH + i], ixs[q], psem[q])

        def ix_wait(q):
            pltpu.make_async_copy(garr.at[0], ixg[q], psem[q]).wait()
            pltpu.make_async_copy(sarr.at[0], ixs[q], psem[q]).wait()

        def g_start(b, q):
            pltpu.async_copy(sin.at[ixg[q]], rows[b], gsem[b])

        def g_wait(b):
            pltpu.make_async_copy(sin.at[ixg[0]], rows[b], gsem[b]).wait()

        def s_start(b, q):
            pltpu.async_copy(rows[b], agg.at[ixs[q]], ssem[b], add=True)

        def s_wait(b):
            pltpu.make_async_copy(rows[b], agg.at[ixs[0]], ssem[b]).wait()

        def _zrow(r, _):
            for j in range(H // L):
                rows[0][r, pl.ds(j * L, L)] = jnp.zeros((L,), _f32)
            return 0

        lax.fori_loop(0, CB, _zrow, 0)
        for h in range(4):
            for t in range(5):
                pltpu.async_copy(
                    rows[0], agg.at[pl.ds(row0 + (h * 5 + t) * CB, CB)],
                    ssem[0])
            for t in range(5):
                pltpu.make_async_copy(
                    rows[0], agg.at[pl.ds(row0, CB)], ssem[0]).wait()
        plsc.subcore_barrier()

        for q in range(GA):
            pltpu.sync_copy(garr.at[w * NCH + q], ixg[q])
            pltpu.sync_copy(sarr.at[s * NCH + q], ixs[q])
        for f in range(GA, IXA):
            ix_start(f, f % NQ)
        for b in range(GA):
            g_start(b, b)
        for i in range(UNR):
            b = i % NB
            ix_start(i + IXA, (i + IXA) % NQ)
            g_wait(b)
            s_start(b, i % NQ)
            if i >= SLAG:
                s_wait((i - SLAG) % NB)
            ix_wait((i + GA) % NQ)
            g_start((i + GA) % NB, (i + GA) % NQ)

        def _gmain(t, _):
            for u in range(UNR):
                i = t * UNR + u
                b = u % NB
                ix_start(i + IXA, (u + IXA) % NQ)
                g_wait(b)
                s_start(b, u % NQ)
                s_wait((u - SLAG) % NB)
                ix_wait((u + GA) % NQ)
                g_start((u + GA) % NB, (u + GA) % NQ)
            return 0

        lax.fori_loop(1, NCH // UNR - 1, _gmain, 0)

        base = NCH - UNR
        for u in range(UNR):
            i = base + u
            b = u % NB
            if i + IXA < NCH:
                ix_start(i + IXA, (u + IXA) % NQ)
            g_wait(b)
            s_start(b, u % NQ)
            s_wait((u - SLAG) % NB)
            if i + GA < NCH:
                ix_wait((u + GA) % NQ)
                g_start((u + GA) % NB, (u + GA) % NQ)
        for u in range(UNR - SLAG, UNR):
            s_wait(u % NB)
        plsc.subcore_barrier()

        for t in range(RS // CB):
            p = t % 2
            db, nb = rows[2 * p], rows[2 * p + 1]
            if t >= 2:
                pltpu.make_async_copy(
                    db, sout.at[pl.ds(coff, CB)], ssem[2 * p]).wait()
            pltpu.async_copy(agg.at[pl.ds(row0 + t * CB, CB)], db, gsem[2 * p])
            pltpu.async_copy(norm2x.at[pl.ds(row0 + t * CB, CB)], nb,
                             gsem[2 * p + 1])
            pltpu.make_async_copy(agg.at[pl.ds(row0, CB)], db,
                                  gsem[2 * p]).wait()
            pltpu.make_async_copy(norm2x.at[pl.ds(row0, CB)], nb,
                                  gsem[2 * p + 1]).wait()

            def _srow(r, _):
                for j in range(H // L):
                    db[r, pl.ds(j * L, L)] = (
                        db[r, pl.ds(j * L, L)] * nb[r, pl.ds(j * L, L)]
                    )
                return 0

            lax.fori_loop(0, CB, _srow, 0)
            pltpu.async_copy(db, sout.at[pl.ds(coff + row0 + t * CB, CB)],
                             ssem[2 * p])
        for p in range(2):
            pltpu.make_async_copy(
                rows[2 * p], sout.at[pl.ds(coff, CB)], ssem[2 * p]).wait()

    souts = [o1, o2]
    sins = [s0] + souts[:1]
    for k in range(2):
        one_hop(sins[k], souts[k])


_hops_call = functools.partial(
    pl.kernel,
    out_type=[jax.ShapeDtypeStruct((NC * NP, H), _f32) for _ in range(2)],
    mesh=plsc.VectorSubcoreMesh(core_axis_name="c", subcore_axis_name="s"),
    scratch_types=(
        [pltpu.VMEM_SHARED((NP, H), _f32)]
        + [pltpu.VMEM((CB, H), _f32) for _ in range(NB)]
        + [pltpu.VMEM((CB,), _i32) for _ in range(NQ)]
        + [pltpu.VMEM((CB,), _i32) for _ in range(NQ)]
        + [pltpu.SemaphoreType.DMA for _ in range(NB)]
        + [pltpu.SemaphoreType.DMA for _ in range(NB)]
        + [pltpu.SemaphoreType.DMA for _ in range(NQ)]
    ),
)(_hops_body)


def _prep_body(feat_ref, deg2_ref, s0_ref, n2x_ref, rn_ref):
    d = deg2_ref[0:NP, :] + deg2_ref[NP : 2 * NP, :]
    cl = jnp.maximum(d, 1.0)
    norm = lax.rsqrt(cl)
    n2x_ref[...] = jnp.broadcast_to(1.0 / cl, (NP, H))
    rn_ref[...] = jnp.sqrt(cl)
    s0_ref[0:NP, :] = feat_ref[:, 0:H] * norm
    s0_ref[NP : 2 * NP, :] = feat_ref[:, H : 2 * H] * norm


_prep_call = pl.pallas_call(
    _prep_body,
    out_shape=[
        jax.ShapeDtypeStruct((NC * NP, H), _f32),
        jax.ShapeDtypeStruct((NP, H), _f32),
        jax.ShapeDtypeStruct((NP, 1), _f32),
    ],
)


def _mm_body(feat_ref, w0_ref, wh_ref, rn_ref, b_ref, *rest):
    s_refs = rest[:8]
    out_ref = rest[8]
    acc = jnp.zeros((RS, D), _f32)
    for k in range(8):
        for c in range(NC):
            acc = acc + jnp.dot(s_refs[k][c], wh_ref[k, c],
                                preferred_element_type=_f32)
    base = jnp.dot(feat_ref[...], w0_ref[...], preferred_element_type=_f32)
    out_ref[...] = base + rn_ref[...] * acc + b_ref[...]


_mm_call = pl.pallas_call(
    _mm_body,
    grid=(NS,),
    in_specs=[
        pl.BlockSpec((RS, D), lambda i: (i, 0)),
        pl.BlockSpec((D, D), lambda i: (0, 0)),
        pl.BlockSpec((8, NC, H, D), lambda i: (0, 0, 0, 0)),
        pl.BlockSpec((RS, 1), lambda i: (i, 0)),
        pl.BlockSpec((1, D), lambda i: (0, 0)),
    ] + [pl.BlockSpec((NC, RS, H), lambda i: (0, i, 0)) for _ in range(8)],
    out_specs=pl.BlockSpec((RS, D), lambda i: (i, 0)),
    out_shape=jax.ShapeDtypeStruct((NP, D), _f32),
)


def kernel(feat, edge_index, W, b):
    src = edge_index[0].astype(_i32)
    dst = edge_index[1].astype(_i32)

    pad = ET - E
    zpad = jnp.zeros((pad,), _i32)
    npad = jnp.full((pad,), N, _i32)
    dst_g = jnp.concatenate([dst, zpad]).reshape(NS * NCH, CB)
    dst_s = jnp.concatenate([dst, npad]).reshape(NS * NCH, CB)
    src_g = jnp.concatenate([src, zpad]).reshape(NS * NCH, CB)
    src_s = jnp.concatenate([src, npad]).reshape(NS * NCH, CB)
    dst_g2 = jnp.concatenate([dst_g, dst_g + NP], axis=0)
    src_g2 = jnp.concatenate([src_g, src_g + NP], axis=0)
    darr = dst_s.reshape(NC * NS, DNCH, DCB)

    featp = jnp.pad(feat, ((0, NP - N), (0, 0)))

    deg2 = _deg_call(darr)
    s0, n2x, rn = _prep_call(featp, deg2.reshape(NC * NP, 1))

    s_a = _hops_call(s0, dst_g2, src_s, n2x)
    s_b = _hops_call(s_a[1], dst_g2, src_s, n2x)
    s_c = _hops_call(s_b[1], src_g2, dst_s, n2x)
    s_d = _hops_call(s_c[1], src_g2, dst_s, n2x)
    s_list = list(s_a) + list(s_b) + list(s_c) + list(s_d)

    w0 = W[0:D]
    wh = W[D:].reshape(8, NC, H, D)
    b2 = b.reshape(1, D)
    s3d = [sk.reshape(NC, NP, H) for sk in s_list]
    outp = _mm_call(featp, w0, wh, rn, b2, *s3d)
    return outp[:N]


# speedup vs baseline: 1.1161x; 1.0148x over previous
import functools

import jax
import jax.numpy as jnp
from jax import lax
from jax.experimental import pallas as pl
from jax.experimental.pallas import tpu as pltpu
from jax.experimental.pallas import tpu_sc as plsc

NC, NS, L = 2, 16, 16
N = 10000
E = 160000
D = 256
H = D // NC
RS = 640
NP = NS * RS
CB = 64
EPT = 10240
NCH = EPT // CB
NB = 4
NQ = 8
GA = 3
SLAG = NB - GA
IXA = GA + 2
UNR = 8
ET = NS * EPT
DCB = 128
EPW = ET // (NC * NS)
DNCH = EPW // DCB

_f32 = jnp.float32
_i32 = jnp.int32


def _deg_body(darr, deg2, degsh, di_all, ones, zb, stg, sem):
    c = lax.axis_index("c")
    s = lax.axis_index("s")
    w = c * NS + s

    for j in range(DCB // L):
        ones[pl.ds(j * L, L)] = jnp.ones((L,), _f32)

    def _z(t, _):
        zb[pl.ds(t * L, L)] = jnp.zeros((L,), _f32)
        return 0

    lax.fori_loop(0, RS // L, _z, 0)
    pltpu.sync_copy(zb, degsh.at[pl.ds(s * RS, RS)])
    pltpu.sync_copy(darr.at[w], di_all)
    plsc.subcore_barrier()

    def _grp(g, _):
        for u in range(8):
            pltpu.async_copy(ones, degsh.at[di_all.at[g * 8 + u]], sem, add=True)
        for u in range(8):
            pltpu.make_async_copy(ones, degsh.at[di_all.at[0]], sem).wait()
        return 0

    lax.fori_loop(0, DNCH // 8, _grp, 0)
    plsc.subcore_barrier()

    pltpu.sync_copy(degsh.at[pl.ds(s * RS, RS)], stg)
    pltpu.sync_copy(stg, deg2.at[pl.ds(c * NP + s * RS, RS)])


_deg_call = functools.partial(
    pl.kernel,
    out_type=jax.ShapeDtypeStruct((NC * NP,), _f32),
    mesh=plsc.VectorSubcoreMesh(core_axis_name="c", subcore_axis_name="s"),
    scratch_types=[
        pltpu.VMEM_SHARED((NP,), _f32),
        pltpu.VMEM((DNCH, DCB), _i32),
        pltpu.VMEM((DCB,), _f32),
        pltpu.VMEM((RS,), _f32),
        pltpu.VMEM((RS,), _f32),
        pltpu.SemaphoreType.DMA,
    ],
)(_deg_body)


def _hops_body(s0, garr, sarr, norm2x,
               o1, o2, o3, o4,
               agg, *rest):
    rows = list(rest[0:NB])
    ixg = list(rest[NB:NB + NQ])
    ixs = list(rest[NB + NQ:NB + 2 * NQ])
    gsem = list(rest[NB + 2 * NQ:2 * NB + 2 * NQ])
    ssem = list(rest[2 * NB + 2 * NQ:3 * NB + 2 * NQ])
    psem = list(rest[3 * NB + 2 * NQ:3 * NB + 3 * NQ])

    c = lax.axis_index("c")
    s = lax.axis_index("s")
    w = c * NS + s
    coff = c * NP
    row0 = s * RS

    def one_hop(sin, sout):
        def ix_start(i, q):
            pltpu.async_copy(garr.at[w * NCH + i], ixg[q], psem[q])
            pltpu.async_copy(sarr.at[s * NCH + i], ixs[q], psem[q])

        def ix_wait(q):
            pltpu.make_async_copy(garr.at[0], ixg[q], psem[q]).wait()
            pltpu.make_async_copy(sarr.at[0], ixs[q], psem[q]).wait()

        def g_start(b, q):
            pltpu.async_copy(sin.at[ixg[q]], rows[b], gsem[b])

        def g_wait(b):
            pltpu.make_async_copy(sin.at[ixg[0]], rows[b], gsem[b]).wait()

        def s_start(b, q):
            pltpu.async_copy(rows[b], agg.at[ixs[q]], ssem[b], add=True)

        def s_wait(b):
            pltpu.make_async_copy(rows[b], agg.at[ixs[0]], ssem[b]).wait()

        def _zrow(r, _):
            for j in range(H // L):
                rows[0][r, pl.ds(j * L, L)] = jnp.zeros((L,), _f32)
            return 0

        lax.fori_loop(0, CB, _zrow, 0)
        for h in range(RS // CB // 5):
            for t in range(5):
                pltpu.async_copy(
                    rows[0], agg.at[pl.ds(row0 + (h * 5 + t) * CB, CB)],
                    ssem[0])
            for t in range(5):
                pltpu.make_async_copy(
                    rows[0], agg.at[pl.ds(row0, CB)], ssem[0]).wait()
        plsc.subcore_barrier()

        for q in range(GA):
            pltpu.sync_copy(garr.at[w * NCH + q], ixg[q])
            pltpu.sync_copy(sarr.at[s * NCH + q], ixs[q])
        for f in range(GA, IXA):
            ix_start(f, f % NQ)
        for b in range(GA):
            g_start(b, b)
        for i in range(UNR):
            b = i % NB
            ix_start(i + IXA, (i + IXA) % NQ)
            g_wait(b)
            s_start(b, i % NQ)
            if i >= SLAG:
                s_wait((i - SLAG) % NB)
            ix_wait((i + GA) % NQ)
            g_start((i + GA) % NB, (i + GA) % NQ)

        def _gmain(t, _):
            for u in range(UNR):
                i = t * UNR + u
                b = u % NB
                ix_start(i + IXA, (u + IXA) % NQ)
                g_wait(b)
                s_start(b, u % NQ)
                s_wait((u - SLAG) % NB)
                ix_wait((u + GA) % NQ)
                g_start((u + GA) % NB, (u + GA) % NQ)
            return 0

        lax.fori_loop(1, NCH // UNR - 1, _gmain, 0)

        base = NCH - UNR
        for u in range(UNR):
            i = base + u
            b = u % NB
            if i + IXA < NCH:
                ix_start(i + IXA, (u + IXA) % NQ)
            g_wait(b)
            s_start(b, u % NQ)
            s_wait((u - SLAG) % NB)
            if i + GA < NCH:
                ix_wait((u + GA) % NQ)
                g_start((u + GA) % NB, (u + GA) % NQ)
        for u in range(UNR - SLAG, UNR):
            s_wait(u % NB)
        plsc.subcore_barrier()

        for t in range(RS // CB):
            p = t % 2
            db, nb = rows[2 * p], rows[2 * p + 1]
            if t >= 2:
                pltpu.make_async_copy(
                    db, sout.at[pl.ds(coff, CB)], ssem[2 * p]).wait()
            pltpu.async_copy(agg.at[pl.ds(row0 + t * CB, CB)], db, gsem[2 * p])
            pltpu.async_copy(norm2x.at[pl.ds(row0 + t * CB, CB)], nb,
                             gsem[2 * p + 1])
            pltpu.make_async_copy(agg.at[pl.ds(row0, CB)], db,
                                  gsem[2 * p]).wait()
            pltpu.make_async_copy(norm2x.at[pl.ds(row0, CB)], nb,
                                  gsem[2 * p + 1]).wait()

            def _srow(r, _):
                for j in range(H // L):
                    db[r, pl.ds(j * L, L)] = (
                        db[r, pl.ds(j * L, L)] * nb[r, pl.ds(j * L, L)]
                    )
                return 0

            lax.fori_loop(0, CB, _srow, 0)
            pltpu.async_copy(db, sout.at[pl.ds(coff + row0 + t * CB, CB)],
                             ssem[2 * p])
        for p in range(2):
            pltpu.make_async_copy(
                rows[2 * p], sout.at[pl.ds(coff, CB)], ssem[2 * p]).wait()

    souts = [o1, o2, o3, o4]
    sins = [s0] + souts[:3]
    for k in range(4):
        one_hop(sins[k], souts[k])


_hops_call = functools.partial(
    pl.kernel,
    out_type=[jax.ShapeDtypeStruct((NC * NP, H), _f32) for _ in range(4)],
    mesh=plsc.VectorSubcoreMesh(core_axis_name="c", subcore_axis_name="s"),
    scratch_types=(
        [pltpu.VMEM_SHARED((NP, H), _f32)]
        + [pltpu.VMEM((CB, H), _f32) for _ in range(NB)]
        + [pltpu.VMEM((CB,), _i32) for _ in range(NQ)]
        + [pltpu.VMEM((CB,), _i32) for _ in range(NQ)]
        + [pltpu.SemaphoreType.DMA for _ in range(NB)]
        + [pltpu.SemaphoreType.DMA for _ in range(NB)]
        + [pltpu.SemaphoreType.DMA for _ in range(NQ)]
    ),
)(_hops_body)


def _prep_body(feat_ref, deg2_ref, s0_ref, n2x_ref, rn_ref):
    d = deg2_ref[0:NP, :] + deg2_ref[NP : 2 * NP, :]
    cl = jnp.maximum(d, 1.0)
    norm = lax.rsqrt(cl)
    n2x_ref[...] = jnp.broadcast_to(1.0 / cl, (NP, H))
    rn_ref[...] = jnp.sqrt(cl)
    s0_ref[0:NP, :] = feat_ref[:, 0:H] * norm
    s0_ref[NP : 2 * NP, :] = feat_ref[:, H : 2 * H] * norm


_prep_call = pl.pallas_call(
    _prep_body,
    out_shape=[
        jax.ShapeDtypeStruct((NC * NP, H), _f32),
        jax.ShapeDtypeStruct((NP, H), _f32),
        jax.ShapeDtypeStruct((NP, 1), _f32),
    ],
)


def _mm_body(feat_ref, w0_ref, wh_ref, rn_ref, b_ref, *rest):
    s_refs = rest[:8]
    out_ref = rest[8]
    acc = jnp.zeros((RS, D), _f32)
    for k in range(8):
        for c in range(NC):
            acc = acc + jnp.dot(s_refs[k][c], wh_ref[k, c],
                                preferred_element_type=_f32)
    base = jnp.dot(feat_ref[...], w0_ref[...], preferred_element_type=_f32)
    out_ref[...] = base + rn_ref[...] * acc + b_ref[...]


_mm_call = pl.pallas_call(
    _mm_body,
    grid=(NS,),
    in_specs=[
        pl.BlockSpec((RS, D), lambda i: (i, 0)),
        pl.BlockSpec((D, D), lambda i: (0, 0)),
        pl.BlockSpec((8, NC, H, D), lambda i: (0, 0, 0, 0)),
        pl.BlockSpec((RS, 1), lambda i: (i, 0)),
        pl.BlockSpec((1, D), lambda i: (0, 0)),
    ] + [pl.BlockSpec((NC, RS, H), lambda i: (0, i, 0)) for _ in range(8)],
    out_specs=pl.BlockSpec((RS, D), lambda i: (i, 0)),
    out_shape=jax.ShapeDtypeStruct((NP, D), _f32),
)


def kernel(feat, edge_index, W, b):
    src = edge_index[0].astype(_i32)
    dst = edge_index[1].astype(_i32)

    pad = ET - E
    zpad = jnp.zeros((pad,), _i32)
    npad = jnp.full((pad,), N, _i32)
    dst_g = jnp.concatenate([dst, zpad]).reshape(NS * NCH, CB)
    dst_s = jnp.concatenate([dst, npad]).reshape(NS * NCH, CB)
    src_g = jnp.concatenate([src, zpad]).reshape(NS * NCH, CB)
    src_s = jnp.concatenate([src, npad]).reshape(NS * NCH, CB)
    dst_g2 = jnp.concatenate([dst_g, dst_g + NP], axis=0)
    src_g2 = jnp.concatenate([src_g, src_g + NP], axis=0)
    darr = dst_s.reshape(NC * NS, DNCH, DCB)

    featp = jnp.pad(feat, ((0, NP - N), (0, 0)))

    deg2 = _deg_call(darr)
    s0, n2x, rn = _prep_call(featp, deg2.reshape(NC * NP, 1))

    s_a = _hops_call(s0, dst_g2, src_s, n2x)
    s_b = _hops_call(s_a[3], src_g2, dst_s, n2x)
    s_list = list(s_a) + list(s_b)

    w0 = W[0:D]
    wh = W[D:].reshape(8, NC, H, D)
    b2 = b.reshape(1, D)
    s3d = [sk.reshape(NC, NP, H) for sk in s_list]
    outp = _mm_call(featp, w0, wh, rn, b2, *s3d)
    return outp[:N]
